# baseline (speedup 1.0000x reference)
import jax
import jax.numpy as jnp
from jax.experimental import pallas as pl
from jax.experimental.pallas import tpu as pltpu

N = 8192
HIDDEN = 32
REG = 1.0
CAPACITY = 3
BLK = 512
B = N // BLK
K = B // 2 + 1


def _tri_kernel(x_ref, w1_ref, b1_ref, w2_ref, b2_ref, out_ref, a_ref, c_ref,
                at_ref):
    i = pl.program_id(0)

    @pl.when(i == 0)
    def _init():
        x = x_ref[:, :]
        acc = jnp.full((B, BLK), b2_ref[0, 0], dtype=jnp.float32)
        for k in range(HIDDEN):
            h = jnp.maximum(x * w1_ref[0, k] + b1_ref[0, k], 0.0)
            acc = acc + w2_ref[0, k] * h
        a = acc * (0.5 / REG)
        a_ref[:, :] = a
        at_ref[:, :] = jnp.repeat(a.T, 128, axis=1)
        c_ref[:, :] = jnp.zeros((B, BLK), jnp.float32)

    ap = at_ref[:, pl.ds(i * 128, 128)]

    acc_t = None
    for k in range(K):
        w = 0.5 if k == K - 1 else 1.0
        jj = jax.lax.rem(i + k, B)
        acol = a_ref[pl.ds(jj, 1), :]
        css = []
        tfold = None
        for c in range(BLK // 128):
            tc = jnp.tanh(ap - acol[:, c * 128:(c + 1) * 128])
            tfold = tc if tfold is None else tfold + tc
            if k > 0:
                css.append(jnp.sum(tc, axis=0).reshape(1, 128))
        if w != 1.0:
            tfold = tfold * w
        acc_t = tfold if k == 0 else acc_t + tfold
        if k > 0:
            cs = jnp.concatenate(css, axis=1)
            if w != 1.0:
                cs = cs * w
            c_ref[pl.ds(jj, 1), :] = c_ref[pl.ds(jj, 1), :] - cs
    u = jnp.sum(acc_t, axis=1).reshape(1, BLK)
    c_ref[pl.ds(i, 1), :] = c_ref[pl.ds(i, 1), :] + u

    @pl.when(i == B - 1)
    def _fin():
        ranks = 0.5 * c_ref[:, :] + (0.5 + 0.5 * N)
        other = ranks % CAPACITY
        out_ref[:, :] = (ranks - other) / CAPACITY + 1.0


@jax.jit
def kernel(x, W1, b1, W2, b2):
    out = pl.pallas_call(
        _tri_kernel,
        grid=(B,),
        in_specs=[
            pl.BlockSpec((B, BLK), lambda i: (0, 0)),
            pl.BlockSpec((1, HIDDEN), lambda i: (0, 0)),
            pl.BlockSpec((1, HIDDEN), lambda i: (0, 0)),
            pl.BlockSpec((1, HIDDEN), lambda i: (0, 0)),
            pl.BlockSpec((1, 1), lambda i: (0, 0)),
        ],
        out_specs=pl.BlockSpec((B, BLK), lambda i: (0, 0)),
        out_shape=jax.ShapeDtypeStruct((B, BLK), jnp.float32),
        scratch_shapes=[
            pltpu.VMEM((B, BLK), jnp.float32),
            pltpu.VMEM((B, BLK), jnp.float32),
            pltpu.VMEM((BLK, B * 128), jnp.float32),
        ],
    )(
        x.reshape(B, BLK),
        W1.reshape(1, HIDDEN),
        b1.reshape(1, HIDDEN),
        W2.reshape(1, HIDDEN),
        b2.reshape(1, 1),
    )
    return out.reshape(N, 1)
